# baseline (speedup 1.0000x reference)
import functools

import jax
import jax.numpy as jnp
from jax import lax
from jax.experimental import pallas as pl
from jax.experimental.pallas import tpu as pltpu
from jax.experimental.pallas import tpu_sc as plsc

B, C, N, R, NE = 16, 2048, 4096, 5, 4
C_TC = 1536
C_SC = C - C_TC
C_BLK = 768
NW = 32
ROWS_PER_W = C_SC // NW
NQ = 4
RB = ROWS_PER_W // NQ
LANES = 16
JU = 4


def _sc_partial_body(x_hbm, w_hbm, out_hbm, w_v, buf_v, acc_v,
                     sem0, sem1, sem2, sem3, sem_out):
    wid = lax.axis_index("s") * 2 + lax.axis_index("c")
    row0 = C_TC + wid * ROWS_PER_W
    pltpu.sync_copy(w_hbm, w_v.at[pl.ds(0, C)])
    wvec = w_v[pl.ds(row0, LANES)]
    sems = [sem0, sem1, sem2, sem3]

    def issue(g, slot):
        b = g // NQ
        q = g % NQ

        @pl.when(b < B)
        def _():
            pltpu.async_copy(
                x_hbm.at[b, pl.ds(row0 + q * RB, RB), pl.ds(0, N)],
                buf_v.at[slot], sems[slot])

    def wait(slot):
        pltpu.make_async_copy(
            x_hbm.at[0, pl.ds(0, RB), pl.ds(0, N)],
            buf_v.at[slot], sems[slot]).wait()

    def wait_out():
        pltpu.make_async_copy(
            acc_v.at[0], out_hbm.at[0, 0], sem_out).wait()

    def consume(slot, grp, aslot, first):
        def jbody(j2, _):
            for u in range(JU):
                j = (j2 * JU + u) * LANES
                a0 = wvec[grp * RB] * buf_v[slot, 0, pl.ds(j, LANES)]
                a1 = wvec[grp * RB + 1] * buf_v[slot, 1, pl.ds(j, LANES)]
                for k in range(2, RB, 2):
                    a0 = a0 + wvec[grp * RB + k] * buf_v[slot, k, pl.ds(j, LANES)]
                    a1 = a1 + wvec[grp * RB + k + 1] * buf_v[slot, k + 1, pl.ds(j, LANES)]
                s = a0 + a1
                if not first:
                    s = s + acc_v[aslot, pl.ds(j, LANES)]
                acc_v[aslot, pl.ds(j, LANES)] = s
            return 0

        lax.fori_loop(0, N // LANES // JU, jbody, 0)

    for g in range(NQ):
        issue(g, g)

    def b2_body(b2, _):
        for sub in range(2):
            b = b2 * 2 + sub
            aslot = sub

            @pl.when(b2 > 0)
            def _():
                wait_out()

            for q in range(NQ):
                wait(q)
                consume(q, q, aslot, q == 0)
                issue((b + 1) * NQ + q, q)
            pltpu.async_copy(acc_v.at[aslot], out_hbm.at[wid, b], sem_out)
        return 0

    lax.fori_loop(0, B // 2, b2_body, 0)
    wait_out()
    wait_out()


_sc_partial = functools.partial(
    pl.kernel,
    _sc_partial_body,
    out_type=jax.ShapeDtypeStruct((NW, B, N), jnp.float32),
    mesh=plsc.VectorSubcoreMesh(core_axis_name="c", subcore_axis_name="s",
                                num_cores=2, num_subcores=16),
    scratch_types=[
        pltpu.VMEM((C + LANES,), jnp.float32),
        pltpu.VMEM((NQ, RB, N), jnp.float32),
        pltpu.VMEM((2, N), jnp.float32),
        pltpu.SemaphoreType.DMA,
        pltpu.SemaphoreType.DMA,
        pltpu.SemaphoreType.DMA,
        pltpu.SemaphoreType.DMA,
        pltpu.SemaphoreType.DMA,
    ],
)()


def _tc_partial_kernel(x_ref, w_ref, out_ref):
    c = pl.program_id(1)
    part = jnp.sum(x_ref[0] * w_ref[...], axis=0, keepdims=True)

    @pl.when(c == 0)
    def _init():
        out_ref[0] = part

    @pl.when(c > 0)
    def _acc():
        out_ref[0] += part


def _combine_kernel(tc_ref, sc_ref, add_ref, cb_ref, W1_ref, b1_ref, W2_ref,
                    b2_ref, Wo_ref, bo_ref, Wl1_ref, bl1_ref, Wl2_ref,
                    bl2_ref, out_ref):
    agg = tc_ref[:, 0, :] + jnp.sum(sc_ref[...], axis=0) + cb_ref[0, 0]
    iota = lax.broadcasted_iota(jnp.int32, (B, N), 1)

    def extract(vals, largest, k):
        out = []
        cur = vals
        fill = -jnp.inf if largest else jnp.inf
        for _ in range(k):
            m = (jnp.max(cur, axis=1, keepdims=True) if largest
                 else jnp.min(cur, axis=1, keepdims=True))
            out.append(m)
            idx = jnp.min(jnp.where(cur == m, iota, N), axis=1, keepdims=True)
            cur = jnp.where(iota == idx, fill, cur)
        return jnp.concatenate(out, axis=1)

    top5 = extract(agg, True, R)
    bot5 = extract(agg, False, R)

    af = add_ref[...]
    feats = jnp.dot(af, Wl1_ref[...], preferred_element_type=jnp.float32)
    prob = jax.nn.sigmoid(feats + bl1_ref[...])
    fl = jnp.dot(prob, Wl2_ref[...],
                 preferred_element_type=jnp.float32) + bl2_ref[...]

    mil = jnp.concatenate([top5, bot5, fl], axis=1)
    h1 = jax.nn.sigmoid(
        jnp.dot(mil, W1_ref[...], preferred_element_type=jnp.float32)
        + b1_ref[...])
    h2 = jax.nn.sigmoid(
        jnp.dot(h1, W2_ref[...], preferred_element_type=jnp.float32)
        + b2_ref[...])
    o = jax.nn.sigmoid(
        jnp.dot(h2, Wo_ref[...], preferred_element_type=jnp.float32)
        + bo_ref[...])
    out_ref[...] = o.reshape(B, 1, 1)


@jax.jit
def _run(in_features, add_features, conv_w, conv_b, W1, b1, W2, b2, Wo, bo,
         Wl1, bl1, Wl2, bl2):
    agg_sc = _sc_partial(in_features, conv_w)

    agg_tc = pl.pallas_call(
        _tc_partial_kernel,
        grid=(B, C_TC // C_BLK),
        in_specs=[
            pl.BlockSpec((1, C_BLK, N), lambda b, c: (b, c, 0)),
            pl.BlockSpec((C_BLK, 1), lambda b, c: (c, 0)),
        ],
        out_specs=pl.BlockSpec((1, 1, N), lambda b, c: (b, 0, 0)),
        out_shape=jax.ShapeDtypeStruct((B, 1, N), jnp.float32),
    )(in_features, conv_w.reshape(C, 1))

    const = lambda *shape: pl.BlockSpec(shape, lambda: (0,) * len(shape))
    return pl.pallas_call(
        _combine_kernel,
        in_specs=[
            const(B, 1, N),
            const(NW, B, N),
            const(B, 3),
            const(1, 1),
            const(2 * R + NE, 200),
            const(1, 200),
            const(200, 100),
            const(1, 100),
            const(100, 1),
            const(1, 1),
            const(3, 4),
            const(1, 4),
            const(4, NE),
            const(1, NE),
        ],
        out_specs=const(B, 1, 1),
        out_shape=jax.ShapeDtypeStruct((B, 1, 1), jnp.float32),
    )(agg_tc, agg_sc, add_features, conv_b.reshape(1, 1), W1,
      b1.reshape(1, 200), W2, b2.reshape(1, 100), Wo, bo.reshape(1, 1),
      Wl1, bl1.reshape(1, 4), Wl2, bl2.reshape(1, NE))


def kernel(in_features, add_features, conv_w, conv_b, W1, b1, W2, b2, Wo, bo,
           Wl1, bl1, Wl2, bl2):
    return _run(in_features, add_features, conv_w, conv_b, W1, b1, W2, b2,
                Wo, bo, Wl1, bl1, Wl2, bl2)


# speedup vs baseline: 1.1298x; 1.1298x over previous
import functools

import jax
import jax.numpy as jnp
from jax.experimental import pallas as pl
from jax.experimental.pallas import tpu as pltpu

B, C, N, R, NE = 16, 2048, 4096, 5, 4
C_BLK = 512
C_STEPS = C // (2 * C_BLK)


def _chowder_kernel(x_ref, x2_ref, add_ref, w_ref, w2_ref, cb_ref, W1_ref,
                    b1_ref, W2_ref, b2_ref, Wo_ref, bo_ref, Wl1_ref, bl1_ref,
                    Wl2_ref, bl2_ref, out_ref, acc_ref):
    c = pl.program_id(1)
    part = (jnp.sum(x_ref[0] * w_ref[...], axis=0, keepdims=True)
            + jnp.sum(x2_ref[0] * w2_ref[...], axis=0, keepdims=True))

    @pl.when(c == 0)
    def _init():
        acc_ref[...] = part + cb_ref[0, 0]

    @pl.when(c > 0)
    def _acc():
        acc_ref[...] += part

    @pl.when(c == C_STEPS - 1)
    def _tail():
        agg = acc_ref[...]
        iota = jax.lax.broadcasted_iota(jnp.int32, (1, N), 1)

        def extract(vals, largest, k):
            out = []
            cur = vals
            fill = -jnp.inf if largest else jnp.inf
            for _ in range(k):
                m = (jnp.max(cur, axis=1, keepdims=True) if largest
                     else jnp.min(cur, axis=1, keepdims=True))
                out.append(m)
                idx = jnp.min(jnp.where(cur == m, iota, N), axis=1,
                              keepdims=True)
                cur = jnp.where(iota == idx, fill, cur)
            return jnp.concatenate(out, axis=1)

        top5 = extract(agg, True, R)
        bot5 = extract(agg, False, R)

        af = add_ref[0]
        feats = jnp.dot(af, Wl1_ref[...], preferred_element_type=jnp.float32)
        prob = jax.nn.sigmoid(feats + bl1_ref[...])
        fl = jnp.dot(prob, Wl2_ref[...],
                     preferred_element_type=jnp.float32) + bl2_ref[...]

        mil = jnp.concatenate([top5, bot5, fl], axis=1)
        h1 = jax.nn.sigmoid(
            jnp.dot(mil, W1_ref[...], preferred_element_type=jnp.float32)
            + b1_ref[...])
        h2 = jax.nn.sigmoid(
            jnp.dot(h1, W2_ref[...], preferred_element_type=jnp.float32)
            + b2_ref[...])
        o = jax.nn.sigmoid(
            jnp.dot(h2, Wo_ref[...], preferred_element_type=jnp.float32)
            + bo_ref[...])
        out_ref[...] = o.reshape(1, 1, 1)


@jax.jit
def _run(in_features, add_features, conv_w, conv_b, W1, b1, W2, b2, Wo, bo,
         Wl1, bl1, Wl2, bl2):
    w2d = conv_w.reshape(C, 1)
    cb = conv_b.reshape(1, 1)
    grid = (B, C_STEPS)
    const = lambda *shape: pl.BlockSpec(shape, lambda b, n: (0,) * len(shape))
    return pl.pallas_call(
        _chowder_kernel,
        grid=grid,
        in_specs=[
            pl.BlockSpec((1, C_BLK, N), lambda b, c: (b, c, 0)),
            pl.BlockSpec((1, C_BLK, N), lambda b, c: (b, C_STEPS + c, 0)),
            pl.BlockSpec((1, 1, 3), lambda b, n: (b, 0, 0)),
            pl.BlockSpec((C_BLK, 1), lambda b, c: (c, 0)),
            pl.BlockSpec((C_BLK, 1), lambda b, c: (C_STEPS + c, 0)),
            const(1, 1),
            const(2 * R + NE, 200),
            const(1, 200),
            const(200, 100),
            const(1, 100),
            const(100, 1),
            const(1, 1),
            const(3, 4),
            const(1, 4),
            const(4, NE),
            const(1, NE),
        ],
        out_specs=pl.BlockSpec((1, 1, 1), lambda b, n: (b, 0, 0)),
        out_shape=jax.ShapeDtypeStruct((B, 1, 1), jnp.float32),
        scratch_shapes=[pltpu.VMEM((1, N), jnp.float32)],
        compiler_params=pltpu.CompilerParams(vmem_limit_bytes=110 * 1024 * 1024),
    )(in_features, in_features, add_features.reshape(B, 1, 3), w2d, w2d, cb,
      W1, b1.reshape(1, 200), W2,
      b2.reshape(1, 100), Wo, bo.reshape(1, 1), Wl1, bl1.reshape(1, 4), Wl2,
      bl2.reshape(1, NE))


def kernel(in_features, add_features, conv_w, conv_b, W1, b1, W2, b2, Wo, bo,
           Wl1, bl1, Wl2, bl2):
    return _run(in_features, add_features, conv_w, conv_b, W1, b1, W2, b2,
                Wo, bo, Wl1, bl1, Wl2, bl2)
